# baseline (speedup 1.0000x reference)
import functools

import jax
import jax.numpy as jnp
from jax import lax
from jax.experimental import pallas as pl
from jax.experimental.pallas import tpu as pltpu
from jax.experimental.pallas import tpu_sc as plsc

_VOCAB = 100000
_D = 64
_B = 4096
_H = 200
_C = 5
_DP = 16

_NC = 2
_NS = 16
_NW = _NC * _NS
_BPW = _B // _NW

_VBLK = 4096
_NSLOT = 4


def _proj_body(tt_ref, w_ref, o_ref):
    prod = lax.dot_general(tt_ref[...], w_ref[...],
                           dimension_numbers=(((0,), (0,)), ((), ())),
                           preferred_element_type=jnp.float32)
    o_ref[:, : _C] = prod


def _project(table_t, w):
    return pl.pallas_call(
        _proj_body,
        grid=(-(-_VOCAB // _VBLK),),
        in_specs=[
            pl.BlockSpec((_D, _VBLK), lambda i: (0, i)),
            pl.BlockSpec((_D, _C), lambda i: (0, 0)),
        ],
        out_specs=pl.BlockSpec((_VBLK, 128), lambda i: (i, 0)),
        out_shape=jax.ShapeDtypeStruct((_VOCAB, 128), jnp.float32),
    )(table_t, w)


_QROWS = 32


def _make_pool():
    mesh = plsc.VectorSubcoreMesh(core_axis_name="c", subcore_axis_name="s")

    @functools.partial(
        pl.kernel,
        mesh=mesh,
        out_type=jax.ShapeDtypeStruct((_B, _DP), jnp.float32),
        scratch_types=[
            pltpu.VMEM((_H // 8, 8, _BPW), jnp.int32),
            pltpu.VMEM((_NSLOT, _QROWS, _DP), jnp.float32),
            pltpu.VMEM((_BPW, _DP), jnp.float32),
            pltpu.VMEM((_DP,), jnp.float32),
            [pltpu.SemaphoreType.DMA] * _NSLOT,
        ],
        compiler_params=pltpu.CompilerParams(use_tc_tiling_on_sc=False),
    )
    def pool(x4_hbm, tw_hbm, bias_hbm, out_hbm, idx_v, rows_v, acc_v, bias_v,
             sems):
        wid = lax.axis_index("s") * _NC + lax.axis_index("c")
        base = wid * _BPW
        pltpu.sync_copy(x4_hbm.at[:, wid], idx_v)
        pltpu.sync_copy(bias_hbm, bias_v)
        bias = bias_v[...]

        def scale_row(j, carry):
            for c in range(_BPW // 16):
                sl = pl.ds(16 * c, 16)
                idx_v[j >> 3, j & 7, sl] = idx_v[j >> 3, j & 7, sl] * 8
            return carry

        lax.fori_loop(0, _H, scale_row, 0)

        def issue(j, s, q):
            pltpu.async_copy(
                tw_hbm.at[idx_v.at[j >> 3, j & 7, pl.ds(_QROWS * q, _QROWS)]],
                rows_v.at[s], sems[s])

        def drain(s):
            pltpu.make_async_copy(
                tw_hbm.at[idx_v.at[0, 0, pl.ds(0, _QROWS)]],
                rows_v.at[s], sems[s]).wait()

        for q in range(_BPW // _QROWS):
            for s in range(_NSLOT):
                issue(s, s, q)

            def group(g, accs, q=q):
                for s in range(_NSLOT):
                    j = _NSLOT * g + s
                    drain(s)
                    accs = tuple(accs[i] + rows_v[s, i]
                                 for i in range(_QROWS))

                    @pl.when(j + _NSLOT < _H)
                    def _():
                        issue(j + _NSLOT, s, q)
                return accs

            accs = lax.fori_loop(0, _H // _NSLOT, group,
                                 (bias,) * _QROWS)
            for i in range(_QROWS):
                acc_v[_QROWS * q + i] = accs[i]

        pltpu.sync_copy(acc_v, out_hbm.at[pl.ds(base, _BPW)])

    return pool


_pool_call = _make_pool()


def kernel(x, table, W, b):
    x = x.astype(jnp.int32)
    bp = jnp.pad(b, (0, _DP - _C))
    tw_pad = _project(table.T, W)
    tw = tw_pad.reshape(8 * _VOCAB, _DP)
    x4 = x.T.reshape(_H // 8, 8, _B // _BPW, _BPW).transpose(0, 2, 1, 3)
    out16 = _pool_call(x4, tw, bp)
    return out16[:, :_C]


# speedup vs baseline: 1.2709x; 1.2709x over previous
import functools

import jax
import jax.numpy as jnp
from jax import lax
from jax.experimental import pallas as pl
from jax.experimental.pallas import tpu as pltpu
from jax.experimental.pallas import tpu_sc as plsc

_VOCAB = 100000
_D = 64
_B = 4096
_H = 200
_C = 5
_DP = 16

_NC = 2
_NS = 16
_NW = _NC * _NS
_BPW = _B // _NW

_VBLK = 4096
_NSLOT = 4


def _proj_body(tt_ref, w_ref, o_ref):
    prod = lax.dot_general(tt_ref[...], w_ref[...],
                           dimension_numbers=(((0,), (0,)), ((), ())),
                           preferred_element_type=jnp.float32)
    o_ref[:, : _C] = prod


def _project(table_t, w):
    return pl.pallas_call(
        _proj_body,
        grid=(-(-_VOCAB // _VBLK),),
        in_specs=[
            pl.BlockSpec((_D, _VBLK), lambda i: (0, i)),
            pl.BlockSpec((_D, _C), lambda i: (0, 0)),
        ],
        out_specs=pl.BlockSpec((_VBLK, 128), lambda i: (i, 0)),
        out_shape=jax.ShapeDtypeStruct((_VOCAB, 128), jnp.float32),
    )(table_t, w)


_QROWS = 16


def _make_pool():
    mesh = plsc.VectorSubcoreMesh(core_axis_name="c", subcore_axis_name="s")

    @functools.partial(
        pl.kernel,
        mesh=mesh,
        out_type=jax.ShapeDtypeStruct((_B, _DP), jnp.float32),
        scratch_types=[
            pltpu.VMEM((_H // 8, 8, _BPW), jnp.int32),
            pltpu.VMEM((_NSLOT, _QROWS, _DP), jnp.float32),
            pltpu.VMEM((_BPW, _DP), jnp.float32),
            pltpu.VMEM((_DP,), jnp.float32),
            pltpu.VMEM_SHARED((_VOCAB, _DP), jnp.float32),
            [pltpu.SemaphoreType.DMA] * _NSLOT,
        ],
        compiler_params=pltpu.CompilerParams(use_tc_tiling_on_sc=False),
    )
    def pool(x4_hbm, tw_hbm, bias_hbm, out_hbm, idx_v, rows_v, acc_v, bias_v,
             dense_s, sems):
        wid = lax.axis_index("s") * _NC + lax.axis_index("c")
        base = wid * _BPW
        pltpu.sync_copy(x4_hbm.at[:, wid], idx_v)
        pltpu.sync_copy(bias_hbm, bias_v)
        bias = bias_v[...]

        sid = lax.axis_index("s")
        vps = _VOCAB // _NS
        pltpu.sync_copy(tw_hbm.at[pl.ds(sid * vps, vps), pl.ds(0, _DP)],
                        dense_s.at[pl.ds(sid * vps, vps)])
        plsc.subcore_barrier()

        def issue(j, s, q):
            pltpu.async_copy(
                dense_s.at[idx_v.at[j >> 3, j & 7, pl.ds(_QROWS * q, _QROWS)]],
                rows_v.at[s], sems[s])

        def drain(s):
            pltpu.make_async_copy(
                dense_s.at[idx_v.at[0, 0, pl.ds(0, _QROWS)]],
                rows_v.at[s], sems[s]).wait()

        for q in range(_BPW // _QROWS):
            for s in range(_NSLOT):
                issue(s, s, q)

            def group(g, accs, q=q):
                for s in range(_NSLOT):
                    j = _NSLOT * g + s
                    drain(s)
                    accs = tuple(accs[i] + rows_v[s, i]
                                 for i in range(_QROWS))

                    @pl.when(j + _NSLOT < _H)
                    def _():
                        issue(j + _NSLOT, s, q)
                return accs

            accs = lax.fori_loop(0, _H // _NSLOT, group,
                                 (bias,) * _QROWS)
            for i in range(_QROWS):
                acc_v[_QROWS * q + i] = accs[i]

        pltpu.sync_copy(acc_v, out_hbm.at[pl.ds(base, _BPW)])

    return pool


_pool_call = _make_pool()


def kernel(x, table, W, b):
    x = x.astype(jnp.int32)
    bp = jnp.pad(b, (0, _DP - _C))
    tw = _project(table.T, W)
    x4 = x.T.reshape(_H // 8, 8, _B // _BPW, _BPW).transpose(0, 2, 1, 3)
    out16 = _pool_call(x4, tw, bp)
    return out16[:, :_C]


# speedup vs baseline: 1.6172x; 1.2725x over previous
import functools

import jax
import jax.numpy as jnp
from jax import lax
from jax.experimental import pallas as pl
from jax.experimental.pallas import tpu as pltpu
from jax.experimental.pallas import tpu_sc as plsc

_VOCAB = 100000
_D = 64
_B = 4096
_H = 200
_C = 5
_DP = 16

_NC = 2
_NS = 16
_NW = _NC * _NS
_BPW = _B // _NW

_VBLK = 4096
_NSLOT = 4


def _proj_body(tt_ref, w_ref, o_ref):
    prod = lax.dot_general(tt_ref[...], w_ref[...],
                           dimension_numbers=(((0,), (0,)), ((), ())),
                           preferred_element_type=jnp.float32)
    o_ref[:, : _C] = prod


def _project(table_t, w):
    return pl.pallas_call(
        _proj_body,
        grid=(-(-_VOCAB // _VBLK),),
        in_specs=[
            pl.BlockSpec((_D, _VBLK), lambda i: (0, i)),
            pl.BlockSpec((_D, _C), lambda i: (0, 0)),
        ],
        out_specs=pl.BlockSpec((_VBLK, 128), lambda i: (i, 0)),
        out_shape=jax.ShapeDtypeStruct((_VOCAB, 128), jnp.float32),
    )(table_t, w)


_NSLOT2 = 8


def _make_pool():
    mesh = plsc.VectorSubcoreMesh(core_axis_name="c", subcore_axis_name="s")

    @functools.partial(
        pl.kernel,
        mesh=mesh,
        out_type=jax.ShapeDtypeStruct((_B, _DP), jnp.float32),
        scratch_types=[
            pltpu.VMEM((3, 8, _BPW), jnp.int32),
            pltpu.VMEM((_NSLOT2, _BPW, _DP), jnp.float32),
            pltpu.VMEM((_BPW, _DP), jnp.float32),
            pltpu.VMEM((_DP,), jnp.float32),
            pltpu.VMEM_SHARED((_VOCAB, _DP), jnp.float32),
            [pltpu.SemaphoreType.DMA] * _NSLOT2,
            pltpu.SemaphoreType.DMA,
        ],
        compiler_params=pltpu.CompilerParams(use_tc_tiling_on_sc=False),
    )
    def pool(x4_hbm, tw_hbm, bias_hbm, out_hbm, idx_v, rows_v, acc_v, bias_v,
             dense_s, sems, isem):
        ngrp = _H // _NSLOT2
        wid = lax.axis_index("s") * _NC + lax.axis_index("c")
        base = wid * _BPW
        pltpu.sync_copy(x4_hbm.at[0, wid], idx_v.at[0])
        pltpu.async_copy(x4_hbm.at[1, wid], idx_v.at[1], isem)
        pltpu.sync_copy(bias_hbm, bias_v)
        bias = bias_v[...]
        for b in range(_BPW):
            acc_v[b] = bias

        sid = lax.axis_index("s")
        vps = _VOCAB // _NS
        pltpu.sync_copy(tw_hbm.at[pl.ds(sid * vps, vps), pl.ds(0, _DP)],
                        dense_s.at[pl.ds(sid * vps, vps)])
        plsc.subcore_barrier()

        def issue(islot, row, s):
            pltpu.async_copy(dense_s.at[idx_v.at[islot, row]],
                             rows_v.at[s], sems[s])

        def drain(s):
            pltpu.make_async_copy(dense_s.at[idx_v.at[0, 0]],
                                  rows_v.at[s], sems[s]).wait()

        for s in range(_NSLOT2):
            issue(0, s, s)

        def group(g, carry):
            @pl.when(g < ngrp - 1)
            def _():
                pltpu.make_async_copy(x4_hbm.at[0, wid], idx_v.at[0],
                                      isem).wait()

            @pl.when(g + 2 < ngrp)
            def _():
                pltpu.async_copy(x4_hbm.at[g + 2, wid],
                                 idx_v.at[lax.rem(g + 2, 3)], isem)

            nslot = lax.rem(g + 1, 3)
            for phase in range(2):
                j0 = _NSLOT2 * g + 4 * phase
                slots = range(4 * phase, 4 * phase + 4)
                for s in slots:
                    drain(s)

                def accum(c, carry, phase=phase):
                    for t in range(16):
                        b = 16 * c + t
                        acc_v[b] = (acc_v[b]
                                    + (rows_v[4 * phase, b]
                                       + rows_v[4 * phase + 1, b])
                                    + (rows_v[4 * phase + 2, b]
                                       + rows_v[4 * phase + 3, b]))
                    return carry

                lax.fori_loop(0, _BPW // 16, accum, 0)

                @pl.when(j0 + _NSLOT2 < _H)
                def _():
                    for s in slots:
                        issue(nslot, s, s)
            return carry

        lax.fori_loop(0, ngrp, group, 0)
        pltpu.sync_copy(acc_v, out_hbm.at[pl.ds(base, _BPW)])

    return pool


_pool_call = _make_pool()


def kernel(x, table, W, b):
    x = x.astype(jnp.int32)
    bp = jnp.pad(b, (0, _DP - _C))
    tw = _project(table.T, W)
    x4 = x.T.reshape(_H // 8, 8, _B // _BPW, _BPW).transpose(0, 2, 1, 3)
    out16 = _pool_call(x4, tw, bp)
    return out16[:, :_C]
